# baseline (speedup 1.0000x reference)
import jax
import jax.numpy as jnp
from jax import lax
from jax.experimental import pallas as pl
from jax.experimental.pallas import tpu as pltpu
from jax.experimental.pallas import tpu_sc as plsc

VOCAB = 1000000
EMBED = 64
WIDE = 128
BATCH = 16384
CTX = 50

NC = 2
NS = 16
NW = NC * NS

ROWS_PER_DMA = 4
CHUNK = ROWS_PER_DMA * CTX
RPW = BATCH // NW
CPW = RPW // ROWS_PER_DMA
NGRP = CPW
NLANE = EMBED // 16
INV_CTX = 1.0 / CTX


def _cbow_body(pidx_hbm, par_hbm, table_hbm, out_hbm,
               pidx_v, par_v, buf_v, out_v, sem0, sem1):
    wid = lax.axis_index("s") * NC + lax.axis_index("c")

    pltpu.sync_copy(pidx_hbm.at[pl.ds(wid * CPW * CHUNK, CPW * CHUNK)], pidx_v)
    pltpu.sync_copy(par_hbm.at[pl.ds(wid * RPW * 2, RPW * 2)], par_v)

    sems = (sem0, sem1)
    iota = lax.broadcasted_iota(jnp.int32, (16,), 0)
    base_q = [iota + 16 * q for q in range(NLANE)]

    def issue(c, parity):
        pltpu.make_async_copy(
            table_hbm.at[pidx_v.at[pl.ds(c * CHUNK, CHUNK)]],
            buf_v.at[parity, 0],
            sems[parity],
        ).start()

    def drain(c, parity):
        pltpu.make_async_copy(
            table_hbm.at[pidx_v.at[pl.ds(c * CHUNK, CHUNK)]],
            buf_v.at[parity, 0],
            sems[parity],
        ).wait()

    def reduce_chunk(g, parity):
        buf = buf_v.at[parity, 0]

        def row_body(rr, carry):
            orow = g * ROWS_PER_DMA + rr
            w0 = plsc.load_gather(par_v, [jnp.full((16,), 2 * orow, jnp.int32)])
            w1 = plsc.load_gather(par_v, [jnp.full((16,), 2 * orow + 1, jnp.int32)])
            acc = [None] * NLANE
            for j in range(CTX):
                w, sh = (w0, j) if j < 32 else (w1, j - 32)
                poff = lax.shift_left(
                    lax.bitwise_and(lax.shift_right_logical(w, sh), 1), 6)
                rvec = jnp.full((16,), rr * CTX + j, jnp.int32)
                for q in range(NLANE):
                    g_q = plsc.load_gather(buf, [rvec, poff + base_q[q]])
                    acc[q] = g_q if acc[q] is None else acc[q] + g_q
            for q in range(NLANE):
                out_v[pl.ds(orow * EMBED + 16 * q, 16)] = acc[q] * INV_CTX
            return carry

        lax.fori_loop(0, ROWS_PER_DMA, row_body, 0, unroll=False)

    issue(0, 0)

    def group_body(g, carry):
        parity = lax.rem(g, 2)

        @pl.when(g + 1 < NGRP)
        def _issue_next():
            nparity = lax.rem(g + 1, 2)

            @pl.when(nparity == 0)
            def _():
                issue(g + 1, 0)

            @pl.when(nparity == 1)
            def _():
                issue(g + 1, 1)

        @pl.when(parity == 0)
        def _p0():
            drain(g, 0)
            reduce_chunk(g, 0)

        @pl.when(parity == 1)
        def _p1():
            drain(g, 1)
            reduce_chunk(g, 1)

        return carry

    lax.fori_loop(0, NGRP, group_body, 0, unroll=False)

    pltpu.sync_copy(out_v, out_hbm.at[pl.ds(wid * RPW * EMBED, RPW * EMBED)])


@jax.jit
def _cbow(pidx, parw, table2):
    mesh = plsc.VectorSubcoreMesh(core_axis_name="c", subcore_axis_name="s")
    f = pl.kernel(
        _cbow_body,
        out_type=jax.ShapeDtypeStruct((BATCH * EMBED,), jnp.float32),
        mesh=mesh,
        scratch_types=[
            pltpu.VMEM((CPW * CHUNK,), jnp.int32),
            pltpu.VMEM((RPW * 2,), jnp.int32),
            pltpu.VMEM((2, 1, CHUNK, WIDE), jnp.float32),
            pltpu.VMEM((RPW * EMBED,), jnp.float32),
            pltpu.SemaphoreType.DMA,
            pltpu.SemaphoreType.DMA,
        ],
        compiler_params=pltpu.CompilerParams(
            use_tc_tiling_on_sc=True, needs_layout_passes=False),
    )
    return f(pidx, parw, table2)


def kernel(inputs, table):
    idx = inputs.astype(jnp.int32)
    pidx = lax.shift_right_logical(idx, 1).reshape(-1)
    par = lax.bitwise_and(idx, 1)
    sh = jnp.arange(32, dtype=jnp.int32)
    w0 = lax.shift_left(par[:, :32], sh[None, :]).sum(axis=1)
    w1 = lax.shift_left(par[:, 32:], sh[None, :CTX - 32]).sum(axis=1)
    parw = jnp.stack([w0, w1], axis=1).reshape(-1)
    table2 = table.reshape(VOCAB // 2, WIDE)
    return _cbow(pidx, parw, table2).reshape(BATCH, EMBED)


# speedup vs baseline: 1.0933x; 1.0933x over previous
import jax
import jax.numpy as jnp
from jax import lax
from jax.experimental import pallas as pl
from jax.experimental.pallas import tpu as pltpu
from jax.experimental.pallas import tpu_sc as plsc

VOCAB = 1000000
EMBED = 64
WIDE = 128
BATCH = 16384
CTX = 50

NC = 2
NS = 16
NW = NC * NS

ROWS_PER_DMA = 4
CHUNK = ROWS_PER_DMA * CTX
RPW = BATCH // NW
CPW = RPW // ROWS_PER_DMA
NGRP = CPW
NLANE = EMBED // 16
INV_CTX = 1.0 / CTX


def _cbow_body(idx_hbm, table_hbm, out_hbm, idx_v, buf_v, out_v, sem0, sem1):
    wid = lax.axis_index("s") * NC + lax.axis_index("c")

    pltpu.sync_copy(idx_hbm.at[pl.ds(wid * CPW * CHUNK, CPW * CHUNK)], idx_v)

    sems = (sem0, sem1)

    def issue(c, parity):
        pltpu.make_async_copy(
            table_hbm.at[idx_v.at[pl.ds(c * CHUNK, CHUNK)]],
            buf_v.at[parity, 0],
            sems[parity],
        ).start()

    def drain(c, parity):
        pltpu.make_async_copy(
            table_hbm.at[idx_v.at[pl.ds(c * CHUNK, CHUNK)]],
            buf_v.at[parity, 0],
            sems[parity],
        ).wait()

    def reduce_chunk(g, parity):
        def row_body(rr, carry):
            j0 = rr * CTX
            acc = [buf_v[parity, 0, j0, pl.ds(16 * q, 16)] for q in range(NLANE)]
            for j in range(1, CTX):
                for q in range(NLANE):
                    acc[q] += buf_v[parity, 0, j0 + j, pl.ds(16 * q, 16)]
            orow = g * ROWS_PER_DMA + rr
            for q in range(NLANE):
                out_v[pl.ds(orow * EMBED + 16 * q, 16)] = acc[q] * INV_CTX
            return carry

        lax.fori_loop(0, ROWS_PER_DMA, row_body, 0, unroll=False)

    issue(0, 0)

    def group_body(g, carry):
        parity = lax.rem(g, 2)

        @pl.when(g + 1 < NGRP)
        def _issue_next():
            nparity = lax.rem(g + 1, 2)

            @pl.when(nparity == 0)
            def _():
                issue(g + 1, 0)

            @pl.when(nparity == 1)
            def _():
                issue(g + 1, 1)

        @pl.when(parity == 0)
        def _p0():
            drain(g, 0)
            reduce_chunk(g, 0)

        @pl.when(parity == 1)
        def _p1():
            drain(g, 1)
            reduce_chunk(g, 1)

        return carry

    lax.fori_loop(0, NGRP, group_body, 0, unroll=False)

    pltpu.sync_copy(out_v, out_hbm.at[pl.ds(wid * RPW * EMBED, RPW * EMBED)])


@jax.jit
def _cbow(idx, tablep):
    mesh = plsc.VectorSubcoreMesh(core_axis_name="c", subcore_axis_name="s")
    f = pl.kernel(
        _cbow_body,
        out_type=jax.ShapeDtypeStruct((BATCH * EMBED,), jnp.float32),
        mesh=mesh,
        scratch_types=[
            pltpu.VMEM((CPW * CHUNK,), jnp.int32),
            pltpu.VMEM((2, 1, CHUNK, WIDE), jnp.float32),
            pltpu.VMEM((RPW * EMBED,), jnp.float32),
            pltpu.SemaphoreType.DMA,
            pltpu.SemaphoreType.DMA,
        ],
        compiler_params=pltpu.CompilerParams(
            use_tc_tiling_on_sc=True, needs_layout_passes=False),
    )
    return f(idx, tablep)


def kernel(inputs, table):
    idx = inputs.astype(jnp.int32).reshape(-1)
    tablep = jnp.pad(table, ((0, 0), (0, WIDE - EMBED)))
    return _cbow(idx, tablep).reshape(BATCH, EMBED)
